# Initial kernel scaffold; baseline (speedup 1.0000x reference)
import jax
import jax.numpy as jnp
from jax.experimental import pallas as pl


def kernel(x, edge_index, batch, W1, b1, W2, b2, Wm1, bm1, Wm2, bm2):
    raise NotImplementedError("write your pallas kernel here")


# speedup vs baseline: 5.1628x; 5.1628x over previous
import functools

import jax
import jax.numpy as jnp
from jax import lax
from jax.experimental import pallas as pl
from jax.experimental.pallas import tpu as pltpu
from jax.experimental.pallas import tpu_sc as plsc

N = 10000
E = 320000
D = 128
G = 64
H = 64
WACC = 80
NC = 2
NS = 16
NW = NC * NS
EPT = E // NW
K = 80
NCHUNK = EPT // K


def _table_body(x_ref, w_ref, b_ref, t_ref):
    t_ref[...] = jnp.dot(x_ref[...], w_ref[...],
                         preferred_element_type=jnp.float32,
                         precision=lax.Precision.HIGHEST) + b_ref[...]


def _make_table(x, wcat, bcat):
    blk = 1000
    return pl.pallas_call(
        _table_body,
        grid=(N // blk,),
        in_specs=[
            pl.BlockSpec((blk, D), lambda i: (i, 0)),
            pl.BlockSpec((D, 2 * H), lambda i: (0, 0)),
            pl.BlockSpec((1, 2 * H), lambda i: (0, 0)),
        ],
        out_specs=pl.BlockSpec((blk, 2 * H), lambda i: (i, 0)),
        out_shape=jax.ShapeDtypeStruct((N, 2 * H), jnp.float32),
    )(x, wcat, bcat)


def _edge_body(tab_hbm, batch_hbm, src_hbm, dst_hbm, out_hbm,
               batch_v, src_v, dst_v, buf_s, buf_d, acc):
    c = lax.axis_index("c")
    s = lax.axis_index("s")
    wid = c * NS + s

    pltpu.sync_copy(batch_hbm, batch_v)
    pltpu.sync_copy(src_hbm.at[wid], src_v)
    pltpu.sync_copy(dst_hbm.at[wid], dst_v)

    zvec = jnp.zeros((16,), jnp.float32)

    def zrow(j, carry):
        for r in range(WACC // 16):
            acc[j, pl.ds(16 * r, 16)] = zvec
        return carry

    lax.fori_loop(0, G, zrow, 0)

    ii = lax.iota(jnp.int32, 16)
    e1 = jnp.where(ii == 0, jnp.float32(1.0), jnp.float32(0.0))

    def chunk_body(ci, carry):
        pltpu.sync_copy(tab_hbm.at[src_v.at[ci]], buf_s)
        pltpu.sync_copy(tab_hbm.at[dst_v.at[ci]], buf_d)

        def grp_body(m, inner):
            srcs16 = src_v[ci, pl.ds(16 * m, 16)]
            g16 = plsc.load_gather(batch_v, [srcs16])
            for jl in range(16):
                j = 16 * m + jl
                g = g16[jl]
                for r in range(H // 16):
                    va = buf_s[j, pl.ds(16 * r, 16)]
                    vb = buf_d[j, pl.ds(H + 16 * r, 16)]
                    plsc.addupdate(acc.at[g, pl.ds(16 * r, 16)],
                                   jnp.maximum(va + vb, 0.0))
                plsc.addupdate(acc.at[g, pl.ds(H, 16)], e1)
            return inner

        lax.fori_loop(0, K // 16, grp_body, 0)
        return carry

    lax.fori_loop(0, NCHUNK, chunk_body, 0)

    pltpu.sync_copy(acc, out_hbm.at[wid])


_edge_kernel = functools.partial(
    pl.kernel,
    out_type=jax.ShapeDtypeStruct((NW, G, WACC), jnp.float32),
    mesh=plsc.VectorSubcoreMesh(core_axis_name="c", subcore_axis_name="s"),
    compiler_params=pltpu.CompilerParams(needs_layout_passes=False),
    scratch_types=[
        pltpu.VMEM((N,), jnp.int32),
        pltpu.VMEM((NCHUNK, K), jnp.int32),
        pltpu.VMEM((NCHUNK, K), jnp.int32),
        pltpu.VMEM((K, 2 * H), jnp.float32),
        pltpu.VMEM((K, 2 * H), jnp.float32),
        pltpu.VMEM((G, WACC), jnp.float32),
    ],
)(_edge_body)


def _head_body(acc_ref, w2_ref, b2_ref, wm1_ref, bm1_ref,
               wm2_ref, bm2_ref, out_ref):
    acc = acc_ref[0]
    for t in range(1, NW):
        acc = acc + acc_ref[t]
    seg = acc[:, :H]
    cnt = acc[:, H:H + 1]
    pooled = jnp.dot(seg, w2_ref[...],
                     preferred_element_type=jnp.float32,
                     precision=lax.Precision.HIGHEST) + cnt * b2_ref[...]
    h2 = jnp.maximum(
        jnp.dot(pooled, wm1_ref[...],
                preferred_element_type=jnp.float32,
                precision=lax.Precision.HIGHEST) + bm1_ref[...], 0.0)
    out_ref[...] = jnp.dot(h2, wm2_ref[...],
                           preferred_element_type=jnp.float32,
                           precision=lax.Precision.HIGHEST) + bm2_ref[...]


def _head(acc, W2, b2, Wm1, bm1, Wm2, bm2):
    return pl.pallas_call(
        _head_body,
        out_shape=jax.ShapeDtypeStruct((G, 8), jnp.float32),
    )(acc, W2, b2.reshape(1, -1), Wm1, bm1.reshape(1, -1),
      Wm2, bm2.reshape(1, -1))


@jax.jit
def kernel(x, edge_index, batch, W1, b1, W2, b2, Wm1, bm1, Wm2, bm2):
    wcat = jnp.concatenate([W1[:D], W1[D:]], axis=1)
    bcat = jnp.concatenate([b1, jnp.zeros((H,), jnp.float32)]).reshape(1, -1)
    tab = _make_table(x, wcat, bcat)
    src3 = edge_index[0].reshape(NW, NCHUNK, K)
    dst3 = edge_index[1].reshape(NW, NCHUNK, K)
    acc = _edge_kernel(tab, batch, src3, dst3)
    return _head(acc, W2, b2, Wm1, bm1, Wm2, bm2)


# speedup vs baseline: 9.6770x; 1.8744x over previous
import functools

import jax
import jax.numpy as jnp
from jax import lax
from jax.experimental import pallas as pl
from jax.experimental.pallas import tpu as pltpu
from jax.experimental.pallas import tpu_sc as plsc

N = 10000
E = 320000
D = 128
G = 64
H = 64
WACC = 80
NC = 2
NS = 16
NW = NC * NS
EPT = E // NW
K = 80
NCHUNK = EPT // K


def _table_body(x_ref, w_ref, b_ref, t_ref):
    t_ref[...] = jnp.dot(x_ref[...], w_ref[...],
                         preferred_element_type=jnp.float32,
                         precision=lax.Precision.HIGHEST) + b_ref[...]


def _make_table(x, wcat, bcat):
    blk = 1000
    return pl.pallas_call(
        _table_body,
        grid=(N // blk,),
        in_specs=[
            pl.BlockSpec((blk, D), lambda i: (i, 0)),
            pl.BlockSpec((D, 2 * H), lambda i: (0, 0)),
            pl.BlockSpec((1, 2 * H), lambda i: (0, 0)),
        ],
        out_specs=pl.BlockSpec((blk, 2 * H), lambda i: (i, 0)),
        out_shape=jax.ShapeDtypeStruct((N, 2 * H), jnp.float32),
    )(x, wcat, bcat)


def _edge_body(tab_hbm, batch_hbm, src_hbm, dst_hbm, out_hbm,
               batch_v, src_v, dst_v, buf_s0, buf_d0, buf_s1, buf_d1,
               acc, sem0, sem1):
    c = lax.axis_index("c")
    s = lax.axis_index("s")
    wid = c * NS + s

    pltpu.sync_copy(batch_hbm, batch_v)
    pltpu.sync_copy(src_hbm.at[wid], src_v)
    pltpu.sync_copy(dst_hbm.at[wid], dst_v)

    zvec = jnp.zeros((16,), jnp.float32)

    def zrow(j, carry):
        for r in range(WACC // 16):
            acc[j, pl.ds(16 * r, 16)] = zvec
        return carry

    lax.fori_loop(0, G, zrow, 0)

    ii = lax.iota(jnp.int32, 16)
    e1 = jnp.where(ii == 0, jnp.float32(1.0), jnp.float32(0.0))

    def gfire(ci, bs, bd, sem):
        pltpu.async_copy(tab_hbm.at[src_v.at[ci]], bs, sem)
        pltpu.async_copy(tab_hbm.at[dst_v.at[ci]], bd, sem)

    def gwait(ci, bs, bd, sem):
        pltpu.make_async_copy(tab_hbm.at[src_v.at[ci]], bs, sem).wait()
        pltpu.make_async_copy(tab_hbm.at[dst_v.at[ci]], bd, sem).wait()

    def compute(ci, bs, bd):
        def grp_body(m, inner):
            srcs16 = src_v[ci, pl.ds(16 * m, 16)]
            g16 = plsc.load_gather(batch_v, [srcs16])
            for jl in range(16):
                j = 16 * m + jl
                g = g16[jl]
                for r in range(H // 16):
                    va = bs[j, pl.ds(16 * r, 16)]
                    vb = bd[j, pl.ds(H + 16 * r, 16)]
                    plsc.addupdate(acc.at[g, pl.ds(16 * r, 16)],
                                   jnp.maximum(va + vb, 0.0))
                plsc.addupdate(acc.at[g, pl.ds(H, 16)], e1)
            return inner

        lax.fori_loop(0, K // 16, grp_body, 0)

    gfire(0, buf_s0, buf_d0, sem0)

    def pair_body(p, carry):
        c0 = 2 * p
        gfire(c0 + 1, buf_s1, buf_d1, sem1)
        gwait(c0, buf_s0, buf_d0, sem0)
        compute(c0, buf_s0, buf_d0)
        gfire(c0 + 2, buf_s0, buf_d0, sem0)
        gwait(c0 + 1, buf_s1, buf_d1, sem1)
        compute(c0 + 1, buf_s1, buf_d1)
        return carry

    lax.fori_loop(0, NCHUNK // 2, pair_body, 0)
    gwait(NCHUNK - 1, buf_s0, buf_d0, sem0)
    compute(NCHUNK - 1, buf_s0, buf_d0)

    pltpu.sync_copy(acc, out_hbm.at[wid])


_edge_kernel = functools.partial(
    pl.kernel,
    out_type=jax.ShapeDtypeStruct((NW, G, WACC), jnp.float32),
    mesh=plsc.VectorSubcoreMesh(core_axis_name="c", subcore_axis_name="s"),
    compiler_params=pltpu.CompilerParams(needs_layout_passes=False),
    scratch_types=[
        pltpu.VMEM((N,), jnp.int32),
        pltpu.VMEM((NCHUNK, K), jnp.int32),
        pltpu.VMEM((NCHUNK, K), jnp.int32),
        pltpu.VMEM((K, 2 * H), jnp.float32),
        pltpu.VMEM((K, 2 * H), jnp.float32),
        pltpu.VMEM((K, 2 * H), jnp.float32),
        pltpu.VMEM((K, 2 * H), jnp.float32),
        pltpu.VMEM((G, WACC), jnp.float32),
        pltpu.SemaphoreType.DMA,
        pltpu.SemaphoreType.DMA,
    ],
)(_edge_body)


def _head_body(acc_ref, w2_ref, b2_ref, wm1_ref, bm1_ref,
               wm2_ref, bm2_ref, out_ref):
    acc = acc_ref[0]
    for t in range(1, NW):
        acc = acc + acc_ref[t]
    seg = acc[:, :H]
    cnt = acc[:, H:H + 1]
    pooled = jnp.dot(seg, w2_ref[...],
                     preferred_element_type=jnp.float32,
                     precision=lax.Precision.HIGHEST) + cnt * b2_ref[...]
    h2 = jnp.maximum(
        jnp.dot(pooled, wm1_ref[...],
                preferred_element_type=jnp.float32,
                precision=lax.Precision.HIGHEST) + bm1_ref[...], 0.0)
    out_ref[...] = jnp.dot(h2, wm2_ref[...],
                           preferred_element_type=jnp.float32,
                           precision=lax.Precision.HIGHEST) + bm2_ref[...]


def _head(acc, W2, b2, Wm1, bm1, Wm2, bm2):
    return pl.pallas_call(
        _head_body,
        out_shape=jax.ShapeDtypeStruct((G, 8), jnp.float32),
    )(acc, W2, b2.reshape(1, -1), Wm1, bm1.reshape(1, -1),
      Wm2, bm2.reshape(1, -1))


@jax.jit
def kernel(x, edge_index, batch, W1, b1, W2, b2, Wm1, bm1, Wm2, bm2):
    wcat = jnp.concatenate([W1[:D], W1[D:]], axis=1)
    bcat = jnp.concatenate([b1, jnp.zeros((H,), jnp.float32)]).reshape(1, -1)
    tab = _make_table(x, wcat, bcat)
    src3 = edge_index[0].reshape(NW, NCHUNK, K)
    dst3 = edge_index[1].reshape(NW, NCHUNK, K)
    acc = _edge_kernel(tab, batch, src3, dst3)
    return _head(acc, W2, b2, Wm1, bm1, Wm2, bm2)


# speedup vs baseline: 13.7978x; 1.4258x over previous
import functools

import jax
import jax.numpy as jnp
from jax import lax
from jax.experimental import pallas as pl
from jax.experimental.pallas import tpu as pltpu
from jax.experimental.pallas import tpu_sc as plsc

N = 10000
E = 320000
D = 128
G = 64
H = 64
WACC = 80
NC = 2
NS = 16
NW = NC * NS
EPT = E // NW
K = 80
NCHUNK = EPT // K


def _bits16(x):
    bf = x.astype(jnp.bfloat16)
    return lax.bitcast_convert_type(bf, jnp.int16).astype(jnp.int32) & 0xFFFF


def _table_body(x_ref, w_ref, b_ref, t_ref):
    ab = jnp.dot(x_ref[...], w_ref[...],
                 preferred_element_type=jnp.float32,
                 precision=lax.Precision.HIGHEST) + b_ref[...]
    a_lo = _bits16(ab[:, 0:32])
    a_hi = _bits16(ab[:, 32:64])
    b_lo = _bits16(ab[:, 64:96])
    b_hi = _bits16(ab[:, 96:128])
    t_ref[...] = jnp.concatenate(
        [a_lo | (a_hi << 16), b_lo | (b_hi << 16)], axis=1)


def _make_table(x, wcat, bcat):
    blk = 1000
    return pl.pallas_call(
        _table_body,
        grid=(N // blk,),
        in_specs=[
            pl.BlockSpec((blk, D), lambda i: (i, 0)),
            pl.BlockSpec((D, 2 * H), lambda i: (0, 0)),
            pl.BlockSpec((1, 2 * H), lambda i: (0, 0)),
        ],
        out_specs=pl.BlockSpec((blk, H), lambda i: (i, 0)),
        out_shape=jax.ShapeDtypeStruct((N, H), jnp.int32),
    )(x, wcat, bcat)


def _edge_body(tab_hbm, batch_hbm, src_hbm, dst_hbm, out_hbm,
               batch_v, src_v, dst_v, buf_s0, buf_d0, buf_s1, buf_d1,
               acc, sem0, sem1):
    c = lax.axis_index("c")
    s = lax.axis_index("s")
    wid = c * NS + s

    pltpu.sync_copy(batch_hbm, batch_v)
    pltpu.sync_copy(src_hbm.at[wid], src_v)
    pltpu.sync_copy(dst_hbm.at[wid], dst_v)

    zvec = jnp.zeros((16,), jnp.float32)

    def zrow(j, carry):
        for r in range(WACC // 16):
            acc[j, pl.ds(16 * r, 16)] = zvec
        return carry

    lax.fori_loop(0, G, zrow, 0)

    ii = lax.iota(jnp.int32, 16)
    e1 = jnp.where(ii == 0, jnp.float32(1.0), jnp.float32(0.0))

    def gfire(ci, bs, bd, sem):
        pltpu.async_copy(tab_hbm.at[src_v.at[ci]], bs, sem)
        pltpu.async_copy(tab_hbm.at[dst_v.at[ci]], bd, sem)

    def gwait(ci, bs, bd, sem):
        pltpu.make_async_copy(tab_hbm.at[src_v.at[ci]], bs, sem).wait()
        pltpu.make_async_copy(tab_hbm.at[dst_v.at[ci]], bd, sem).wait()

    def compute(ci, bs, bd):
        hmask = jnp.int32(-65536)
        zero = jnp.zeros((16,), jnp.float32)

        def grp_body(m, inner):
            srcs16 = src_v[ci, pl.ds(16 * m, 16)]
            g16 = plsc.load_gather(batch_v, [srcs16])
            for jl in range(16):
                j = 16 * m + jl
                g = g16[jl]
                for r in range(2):
                    va = bs[j, pl.ds(16 * r, 16)]
                    vb = bd[j, pl.ds(32 + 16 * r, 16)]
                    alo = plsc.bitcast(va << 16, jnp.float32)
                    blo = plsc.bitcast(vb << 16, jnp.float32)
                    ahi = plsc.bitcast(va & hmask, jnp.float32)
                    bhi = plsc.bitcast(vb & hmask, jnp.float32)
                    hlo = jnp.maximum(alo + blo, zero)
                    hhi = jnp.maximum(ahi + bhi, zero)
                    plsc.addupdate(acc.at[g, pl.ds(16 * r, 16)], hlo)
                    plsc.addupdate(acc.at[g, pl.ds(32 + 16 * r, 16)], hhi)
                plsc.addupdate(acc.at[g, pl.ds(H, 16)], e1)
            return inner

        lax.fori_loop(0, K // 16, grp_body, 0)

    gfire(0, buf_s0, buf_d0, sem0)

    def pair_body(p, carry):
        c0 = 2 * p
        gfire(c0 + 1, buf_s1, buf_d1, sem1)
        gwait(c0, buf_s0, buf_d0, sem0)
        compute(c0, buf_s0, buf_d0)
        gfire(c0 + 2, buf_s0, buf_d0, sem0)
        gwait(c0 + 1, buf_s1, buf_d1, sem1)
        compute(c0 + 1, buf_s1, buf_d1)
        return carry

    lax.fori_loop(0, NCHUNK // 2, pair_body, 0)
    gwait(NCHUNK - 1, buf_s0, buf_d0, sem0)
    compute(NCHUNK - 1, buf_s0, buf_d0)

    pltpu.sync_copy(acc, out_hbm.at[wid])


_edge_kernel = functools.partial(
    pl.kernel,
    out_type=jax.ShapeDtypeStruct((NW, G, WACC), jnp.float32),
    mesh=plsc.VectorSubcoreMesh(core_axis_name="c", subcore_axis_name="s"),
    compiler_params=pltpu.CompilerParams(needs_layout_passes=False,
                                         use_tc_tiling_on_sc=False),
    scratch_types=[
        pltpu.VMEM((N,), jnp.int32),
        pltpu.VMEM((NCHUNK, K), jnp.int32),
        pltpu.VMEM((NCHUNK, K), jnp.int32),
        pltpu.VMEM((K, H), jnp.int32),
        pltpu.VMEM((K, H), jnp.int32),
        pltpu.VMEM((K, H), jnp.int32),
        pltpu.VMEM((K, H), jnp.int32),
        pltpu.VMEM((G, WACC), jnp.float32),
        pltpu.SemaphoreType.DMA,
        pltpu.SemaphoreType.DMA,
    ],
)(_edge_body)


def _head_body(acc_ref, w2_ref, b2_ref, wm1_ref, bm1_ref,
               wm2_ref, bm2_ref, out_ref):
    acc = acc_ref[0]
    for t in range(1, NW):
        acc = acc + acc_ref[t]
    seg = acc[:, :H]
    cnt = acc[:, H:H + 1]
    pooled = jnp.dot(seg, w2_ref[...],
                     preferred_element_type=jnp.float32,
                     precision=lax.Precision.HIGHEST) + cnt * b2_ref[...]
    h2 = jnp.maximum(
        jnp.dot(pooled, wm1_ref[...],
                preferred_element_type=jnp.float32,
                precision=lax.Precision.HIGHEST) + bm1_ref[...], 0.0)
    out_ref[...] = jnp.dot(h2, wm2_ref[...],
                           preferred_element_type=jnp.float32,
                           precision=lax.Precision.HIGHEST) + bm2_ref[...]


def _head(acc, W2, b2, Wm1, bm1, Wm2, bm2):
    return pl.pallas_call(
        _head_body,
        out_shape=jax.ShapeDtypeStruct((G, 8), jnp.float32),
    )(acc, W2, b2.reshape(1, -1), Wm1, bm1.reshape(1, -1),
      Wm2, bm2.reshape(1, -1))


@jax.jit
def kernel(x, edge_index, batch, W1, b1, W2, b2, Wm1, bm1, Wm2, bm2):
    wcat = jnp.concatenate([W1[:D], W1[D:]], axis=1)
    bcat = jnp.concatenate([b1, jnp.zeros((H,), jnp.float32)]).reshape(1, -1)
    tab = _make_table(x, wcat, bcat)
    src3 = edge_index[0].reshape(NW, NCHUNK, K)
    dst3 = edge_index[1].reshape(NW, NCHUNK, K)
    acc = _edge_kernel(tab, batch, src3, dst3)
    return _head(acc, W2, b2, Wm1, bm1, Wm2, bm2)
